# baseline (speedup 1.0000x reference)
import functools

import jax
import jax.numpy as jnp
from jax import lax
from jax.experimental import pallas as pl
from jax.experimental.pallas import tpu as pltpu
from jax.experimental.pallas import tpu_sc as plsc

F32 = jnp.float32
HI = lax.Precision.HIGHEST

D = 768
T = 2048
KH = 32
NH = 12
DH = 64
LD = 64
VOCAB = 32128
QCH = 128
NQC = T // QCH


def _dot(a, b, prec=HI):
    return lax.dot_general(a, b, (((1,), (0,)), ((), ())),
                           preferred_element_type=F32, precision=prec)


def _dot_nt(a, b, prec=HI):
    return lax.dot_general(a, b, (((1,), (1,)), ((), ())),
                           preferred_element_type=F32, precision=prec)


def _ln(x):
    mu = jnp.mean(x, axis=-1, keepdims=True)
    var = jnp.mean((x - mu) ** 2, axis=-1, keepdims=True)
    return (x - mu) / jnp.sqrt(var + 1e-5)


def _sigmoid(x):
    return 1.0 / (1.0 + jnp.exp(-x))


def _softmax(lg):
    m = jnp.max(lg, axis=-1, keepdims=True)
    e = jnp.exp(lg - m)
    return e / jnp.sum(e, axis=-1, keepdims=True)


def _chunked_store(o_ref, fn, n=8):
    rows = T // n

    def body(i, carry):
        sl = pl.ds(i * rows, rows)
        o_ref[sl, :] = fn(sl)
        return carry

    lax.fori_loop(0, n, body, 0)


RCH = 128


def _route(s_col, sc_s, rank_s):
    sc_s[:] = s_col
    s_row = lax.transpose(s_col, (1, 0))
    col_i = lax.broadcasted_iota(jnp.int32, (1, T), 1)

    def body(i, carry):
        sl = pl.ds(i * RCH, RCH)
        sc = sc_s[sl, :]
        row_i = lax.broadcasted_iota(jnp.int32, (RCH, 1), 0) + i * RCH
        gt = s_row > sc
        tie = (s_row == sc) & (col_i < row_i)
        rank_s[sl, :] = jnp.sum((gt | tie).astype(F32), axis=1,
                                keepdims=True)
        return carry

    lax.fori_loop(0, T // RCH, body, 0)
    rank_col = rank_s[:].astype(jnp.int32)
    rank_row = lax.transpose(rank_col, (1, 0))
    oh = jnp.where(
        rank_row == lax.broadcasted_iota(jnp.int32, (KH, T), 0), 1.0, 0.0)
    oht = jnp.where(
        rank_col == lax.broadcasted_iota(jnp.int32, (T, KH), 1), 1.0, 0.0)
    return oh, oht


def _heavy_mha(q, k, v, wo, qv, mask):
    outs = []
    for hd in range(NH):
        sl = slice(hd * DH, (hd + 1) * DH)
        lg = _dot_nt(q[:, sl], k[:, sl]) * (DH ** -0.5)
        if mask is not None:
            lg = jnp.where(mask, lg, jnp.float32(-1e9))
        outs.append(_dot(_softmax(lg), v[:, sl]))
    out = jnp.concatenate(outs, axis=-1)
    return _dot(out, wo) * _sigmoid(qv)


def _heavy(h_ref, rq_col, rkv_col, wq, wk, wv, wo, causal, sc_s, rank_s):
    s2 = _dot(h_ref[:], jnp.concatenate([rq_col, rkv_col], axis=1))
    sq0 = s2[:, 0:1]
    skv0 = s2[:, 1:2]
    oh_q, oht_q = _route(sq0, sc_s, rank_s)
    oh_k, oht_k = _route(skv0, sc_s, rank_s)
    qv = _dot(oh_q, sq0)
    kvv = _dot(oh_k, skv0)
    gs = _dot(jnp.concatenate([oh_q, oh_k], axis=0), h_ref[:])
    qs = gs[:KH]
    ks = gs[KH:] * _sigmoid(kvv)
    mask = None
    if causal:
        pos_col = lax.broadcasted_iota(jnp.int32, (T, 1), 0).astype(F32)
        pos_row = lax.broadcasted_iota(jnp.int32, (1, T), 1).astype(F32)
        qi = _dot(oh_q, pos_col)
        kvi = _dot(pos_row, oht_k)
        mask = kvi <= qi
    hout = _heavy_mha(_dot(qs, wq), _dot(ks, wk), _dot(ks, wv),
                      wo, qv, mask)
    return oht_q, hout


def _heavy_cross(h_ref, ks, rq_col, wq, wk, wv, wo, sc_s, rank_s):
    sq0 = _dot(h_ref[:], rq_col)
    oh_q, oht_q = _route(sq0, sc_s, rank_s)
    qv = _dot(oh_q, sq0)
    qs = _dot(oh_q, h_ref[:])
    hout = _heavy_mha(_dot(qs, wq), _dot(ks, wk), _dot(ks, wv),
                      wo, qv, None)
    return oht_q, hout


def _light_attn(q_s, k_s, v_s, ao_s, causal):
    def body(i, carry):
        qc = q_s[pl.ds(i * QCH, QCH), :]
        lg = _dot_nt(qc, k_s[:]) * (LD ** -0.5)
        if causal:
            row = lax.broadcasted_iota(jnp.int32, (QCH, T), 0) + i * QCH
            col = lax.broadcasted_iota(jnp.int32, (QCH, T), 1)
            lg = jnp.where(col <= row, lg, jnp.float32(-1e9))
        ao_s[pl.ds(i * QCH, QCH), :] = _dot(_softmax(lg), v_s[:])
        return carry

    lax.fori_loop(0, NQC, body, 0)


def _fill_qkv(src_ref, ws, outs):
    wcat = jnp.concatenate(ws, axis=1)

    def body(i, carry):
        sl = pl.ds(i * (T // 8), T // 8)
        prj = _dot(_ln(src_ref[sl, :]), wcat)
        off = 0
        for o, w_ in zip(outs, ws):
            o[sl, :] = prj[:, off:off + w_.shape[1]]
            off += w_.shape[1]
        return carry

    lax.fori_loop(0, 8, body, 0)


def _light_body(causal, cross, x_ref, *args):
    if cross:
        (enc_ref, lq, lk, lv, lo, o_ref, q_s, k_s, v_s, ao_s) = args
        _fill_qkv(x_ref, [lq[:]], [q_s])
        _fill_qkv(enc_ref, [lk[:], lv[:]], [k_s, v_s])
    else:
        (lq, lk, lv, lo, o_ref, q_s, k_s, v_s, ao_s) = args
        _fill_qkv(x_ref, [lq[:], lk[:], lv[:]], [q_s, k_s, v_s])
    _light_attn(q_s, k_s, v_s, ao_s, causal)
    _chunked_store(o_ref, lambda sl: x_ref[sl, :] + _dot(ao_s[sl, :], lo[:]))


def _heavy_body(causal, x_ref, a_ref, rq, rkv, hq, hk, hv, ho, o_ref,
                oht_s, sc_s, rank_s, h_s):
    _chunked_store(h_s, lambda sl: _ln(x_ref[sl, :]))
    oht_q, hout = _heavy(h_s, rq[:], rkv[:],
                         hq[:], hk[:], hv[:], ho[:], causal, sc_s, rank_s)
    oht_s[:] = oht_q
    _chunked_store(o_ref,
                   lambda sl: a_ref[sl, :] + _dot(oht_s[sl, :], hout))


def _kvprep_body(enc_ref, rkv, o_ref, sc_s, rank_s, e_s):
    _chunked_store(e_s, lambda sl: _ln(enc_ref[sl, :]))
    skv0 = _dot(e_s[:], rkv[:])
    oh_k, _ = _route(skv0, sc_s, rank_s)
    kvv = _dot(oh_k, skv0)
    o_ref[:] = _dot(oh_k, e_s[:]) * _sigmoid(kvv)


def _cross_heavy_body(x_ref, a_ref, ks_ref, rq, hq, hk, hv, ho, o_ref,
                      oht_s, sc_s, rank_s, h_s):
    _chunked_store(h_s, lambda sl: _ln(x_ref[sl, :]))
    oht_q, hout = _heavy_cross(h_s, ks_ref[:], rq[:],
                               hq[:], hk[:], hv[:], ho[:], sc_s, rank_s)
    oht_s[:] = oht_q
    _chunked_store(o_ref,
                   lambda sl: a_ref[sl, :] + _dot(oht_s[sl, :], hout))


def _ff_light_body(x_ref, lf1, lf2, o_ref):
    _chunked_store(
        o_ref,
        lambda sl: x_ref[sl, :] + _dot(
            jax.nn.gelu(_dot(_ln(x_ref[sl, :]), lf1[:])), lf2[:]))


def _ff_heavy_body(x_ref, y_ref, rf, hf1, hf2, o_ref, oht_s, sc_s, rank_s,
                   h_s):
    _chunked_store(h_s, lambda sl: _ln(x_ref[sl, :]))
    fs0 = _dot(h_s[:], rf[:])
    oh, oht = _route(fs0, sc_s, rank_s)
    fv = _dot(oh, fs0)
    sel = _dot(oh, h_s[:])
    hff = _dot(jax.nn.gelu(_dot(sel, hf1[:])), hf2[:]) * _sigmoid(fv)
    oht_s[:] = oht
    _chunked_store(o_ref,
                   lambda sl: y_ref[sl, :] + _dot(oht_s[sl, :], hff))


_TC_PARAMS = pltpu.CompilerParams(vmem_limit_bytes=128 * 1024 * 1024)
_OUT_TD = jax.ShapeDtypeStruct((T, D), F32)


_LIGHT_SCRATCH = [pltpu.VMEM((T, LD), F32) for _ in range(4)]
_HEAVY_SCRATCH = [pltpu.VMEM((T, KH), F32),
                  pltpu.VMEM((T, 1), F32), pltpu.VMEM((T, 1), F32),
                  pltpu.VMEM((T, D), F32)]


def _self_block(x, p, causal):
    a = pl.pallas_call(
        functools.partial(_light_body, causal, False),
        out_shape=_OUT_TD, scratch_shapes=list(_LIGHT_SCRATCH),
        compiler_params=_TC_PARAMS)(
        x, p['lq'], p['lk'], p['lv'], p['lo'])
    return pl.pallas_call(
        functools.partial(_heavy_body, causal),
        out_shape=_OUT_TD, scratch_shapes=list(_HEAVY_SCRATCH),
        compiler_params=_TC_PARAMS)(
        x, a, p['rq'].reshape(D, 1), p['rkv'].reshape(D, 1),
        p['hq'], p['hk'], p['hv'], p['ho'])


def _cross_block(x, enc, p):
    a = pl.pallas_call(
        functools.partial(_light_body, False, True),
        out_shape=_OUT_TD, scratch_shapes=list(_LIGHT_SCRATCH),
        compiler_params=_TC_PARAMS)(
        x, enc, p['clq'], p['clk'], p['clv'], p['clo'])
    ks = pl.pallas_call(
        _kvprep_body, out_shape=jax.ShapeDtypeStruct((KH, D), F32),
        scratch_shapes=[pltpu.VMEM((T, 1), F32), pltpu.VMEM((T, 1), F32),
                        pltpu.VMEM((T, D), F32)],
        compiler_params=_TC_PARAMS)(
        enc, p['crkv'].reshape(D, 1))
    return pl.pallas_call(
        _cross_heavy_body,
        out_shape=_OUT_TD, scratch_shapes=list(_HEAVY_SCRATCH),
        compiler_params=_TC_PARAMS)(
        x, a, ks, p['crq'].reshape(D, 1),
        p['cq'], p['ck'], p['cv'], p['co'])


def _cond_ff(x, p):
    y = pl.pallas_call(
        _ff_light_body, out_shape=_OUT_TD,
        compiler_params=_TC_PARAMS)(x, p['lf1'], p['lf2'])
    return pl.pallas_call(
        _ff_heavy_body, out_shape=_OUT_TD,
        scratch_shapes=[pltpu.VMEM((T, KH), F32),
                        pltpu.VMEM((T, 1), F32), pltpu.VMEM((T, 1), F32),
                        pltpu.VMEM((T, D), F32)],
        compiler_params=_TC_PARAMS)(
        x, y, p['rf'].reshape(D, 1), p['hf1'], p['hf2'])


def _lm_head_body(x_ref, w_ref, b_ref, o_ref):
    o_ref[:] = jnp.dot(x_ref[:], w_ref[:],
                       preferred_element_type=F32) + b_ref[:]


def _lm_head(y, w, b):
    return pl.pallas_call(
        _lm_head_body,
        grid=(VOCAB // 128,),
        in_specs=[pl.BlockSpec((T, D), lambda i: (0, 0)),
                  pl.BlockSpec((D, 128), lambda i: (0, i)),
                  pl.BlockSpec((1, 128), lambda i: (0, i))],
        out_specs=pl.BlockSpec((T, 128), lambda i: (0, i)),
        out_shape=jax.ShapeDtypeStruct((T, VOCAB), F32),
        compiler_params=_TC_PARAMS,
    )(y, w, b.reshape(1, VOCAB))


def _embed_gather(table, ids):
    info = plsc.get_sparse_core_info()
    nc, ns = info.num_cores, info.num_subcores
    nw = nc * ns
    bpw = T // nw
    mesh = plsc.VectorSubcoreMesh(core_axis_name="c", subcore_axis_name="s")

    @functools.partial(
        pl.kernel, mesh=mesh,
        out_type=jax.ShapeDtypeStruct((T, D), F32),
        scratch_types=[pltpu.VMEM((bpw,), jnp.int32),
                       pltpu.VMEM((bpw, D), F32),
                       pltpu.SemaphoreType.DMA])
    def k(table_hbm, idx_hbm, out_hbm, idx_v, rows_v, sem):
        wid = lax.axis_index("s") * nc + lax.axis_index("c")
        base = wid * bpw
        pltpu.sync_copy(idx_hbm.at[pl.ds(base, bpw)], idx_v)
        pltpu.async_copy(table_hbm.at[idx_v], rows_v, sem).wait()
        pltpu.sync_copy(rows_v, out_hbm.at[pl.ds(base, bpw)])

    return k(table, ids)


def kernel(params, input_ids, decoder_input_ids):
    p = params
    x = _embed_gather(p['enc_embed'], input_ids.reshape(T))
    for lp in p['enc_layers']:
        x = _self_block(x, lp, causal=False)
        x = _cond_ff(x, lp)
    enc = x
    y = _embed_gather(p['dec_embed'], decoder_input_ids.reshape(T))
    for lp in p['dec_layers']:
        y = _self_block(y, lp, causal=True)
        y = _cross_block(y, enc, lp)
        y = _cond_ff(y, lp)
    logits = _lm_head(y, p['lm_w'], p['lm_b'])
    return logits.reshape(1, T, VOCAB)


# speedup vs baseline: 1.3054x; 1.3054x over previous
import functools

import jax
import jax.numpy as jnp
from jax import lax
from jax.experimental import pallas as pl
from jax.experimental.pallas import tpu as pltpu
from jax.experimental.pallas import tpu_sc as plsc

F32 = jnp.float32
HI = lax.Precision.HIGHEST
MED = lax.Precision.DEFAULT

D = 768
T = 2048
KH = 32
NH = 12
DH = 64
LD = 64
VOCAB = 32128
QCH = 128
NQC = T // QCH


def _dot(a, b, prec=MED):
    return lax.dot_general(a, b, (((1,), (0,)), ((), ())),
                           preferred_element_type=F32, precision=prec)


def _dot_nt(a, b, prec=MED):
    return lax.dot_general(a, b, (((1,), (1,)), ((), ())),
                           preferred_element_type=F32, precision=prec)


def _ln(x):
    mu = jnp.mean(x, axis=-1, keepdims=True)
    var = jnp.mean((x - mu) ** 2, axis=-1, keepdims=True)
    return (x - mu) / jnp.sqrt(var + 1e-5)


def _sigmoid(x):
    return 1.0 / (1.0 + jnp.exp(-x))


def _softmax(lg):
    m = jnp.max(lg, axis=-1, keepdims=True)
    e = jnp.exp(lg - m)
    return e / jnp.sum(e, axis=-1, keepdims=True)


def _chunked_store(o_ref, fn, n=8):
    rows = T // n

    def body(i, carry):
        sl = pl.ds(i * rows, rows)
        o_ref[sl, :] = fn(sl)
        return carry

    lax.fori_loop(0, n, body, 0)


RCH = 128


def _route(s_col, sc_s, rank_s):
    sc_s[:] = s_col
    s_row = lax.transpose(s_col, (1, 0))
    col_i = lax.broadcasted_iota(jnp.int32, (1, T), 1)

    def body(i, carry):
        sl = pl.ds(i * RCH, RCH)
        sc = sc_s[sl, :]
        row_i = lax.broadcasted_iota(jnp.int32, (RCH, 1), 0) + i * RCH
        gt = s_row > sc
        tie = (s_row == sc) & (col_i < row_i)
        rank_s[sl, :] = jnp.sum((gt | tie).astype(F32), axis=1,
                                keepdims=True)
        return carry

    lax.fori_loop(0, T // RCH, body, 0)
    rank_col = rank_s[:].astype(jnp.int32)
    rank_row = lax.transpose(rank_col, (1, 0))
    oh = jnp.where(
        rank_row == lax.broadcasted_iota(jnp.int32, (KH, T), 0), 1.0, 0.0)
    oht = jnp.where(
        rank_col == lax.broadcasted_iota(jnp.int32, (T, KH), 1), 1.0, 0.0)
    return oh, oht


def _heavy_mha(q, k, v, wo, qv, mask):
    outs = []
    for hd in range(NH):
        sl = slice(hd * DH, (hd + 1) * DH)
        lg = _dot_nt(q[:, sl], k[:, sl]) * (DH ** -0.5)
        if mask is not None:
            lg = jnp.where(mask, lg, jnp.float32(-1e9))
        outs.append(_dot(_softmax(lg), v[:, sl]))
    out = jnp.concatenate(outs, axis=-1)
    return _dot(out, wo) * _sigmoid(qv)


def _heavy(h_ref, rq_col, rkv_col, wq, wk, wv, wo, causal, sc_s, rank_s):
    s2 = _dot(h_ref[:], jnp.concatenate([rq_col, rkv_col], axis=1), HI)
    sq0 = s2[:, 0:1]
    skv0 = s2[:, 1:2]
    oh_q, oht_q = _route(sq0, sc_s, rank_s)
    oh_k, oht_k = _route(skv0, sc_s, rank_s)
    qv = _dot(oh_q, sq0, HI)
    kvv = _dot(oh_k, skv0, HI)
    gs = _dot(jnp.concatenate([oh_q, oh_k], axis=0), h_ref[:], HI)
    qs = gs[:KH]
    ks = gs[KH:] * _sigmoid(kvv)
    mask = None
    if causal:
        pos_col = lax.broadcasted_iota(jnp.int32, (T, 1), 0).astype(F32)
        pos_row = lax.broadcasted_iota(jnp.int32, (1, T), 1).astype(F32)
        qi = _dot(oh_q, pos_col, HI)
        kvi = _dot(pos_row, oht_k, HI)
        mask = kvi <= qi
    hout = _heavy_mha(_dot(qs, wq), _dot(ks, wk), _dot(ks, wv),
                      wo, qv, mask)
    return oht_q, hout


def _heavy_cross(h_ref, ks, rq_col, wq, wk, wv, wo, sc_s, rank_s):
    sq0 = _dot(h_ref[:], rq_col, HI)
    oh_q, oht_q = _route(sq0, sc_s, rank_s)
    qv = _dot(oh_q, sq0, HI)
    qs = _dot(oh_q, h_ref[:], HI)
    hout = _heavy_mha(_dot(qs, wq), _dot(ks, wk), _dot(ks, wv),
                      wo, qv, None)
    return oht_q, hout


def _light_attn(q_s, k_s, v_s, ao_s, causal):
    def body(i, carry):
        qc = q_s[pl.ds(i * QCH, QCH), :]
        lg = _dot_nt(qc, k_s[:]) * (LD ** -0.5)
        if causal:
            row = lax.broadcasted_iota(jnp.int32, (QCH, T), 0) + i * QCH
            col = lax.broadcasted_iota(jnp.int32, (QCH, T), 1)
            lg = jnp.where(col <= row, lg, jnp.float32(-1e9))
        ao_s[pl.ds(i * QCH, QCH), :] = _dot(_softmax(lg), v_s[:])
        return carry

    lax.fori_loop(0, NQC, body, 0)


def _fill_qkv(src_ref, ws, outs):
    wcat = jnp.concatenate(ws, axis=1)

    def body(i, carry):
        sl = pl.ds(i * (T // 8), T // 8)
        prj = _dot(_ln(src_ref[sl, :]), wcat)
        off = 0
        for o, w_ in zip(outs, ws):
            o[sl, :] = prj[:, off:off + w_.shape[1]]
            off += w_.shape[1]
        return carry

    lax.fori_loop(0, 8, body, 0)


def _light_body(causal, cross, x_ref, *args):
    if cross:
        (enc_ref, lq, lk, lv, lo, o_ref, q_s, k_s, v_s, ao_s) = args
        _fill_qkv(x_ref, [lq[:]], [q_s])
        _fill_qkv(enc_ref, [lk[:], lv[:]], [k_s, v_s])
    else:
        (lq, lk, lv, lo, o_ref, q_s, k_s, v_s, ao_s) = args
        _fill_qkv(x_ref, [lq[:], lk[:], lv[:]], [q_s, k_s, v_s])
    _light_attn(q_s, k_s, v_s, ao_s, causal)
    _chunked_store(o_ref, lambda sl: x_ref[sl, :] + _dot(ao_s[sl, :], lo[:]))


def _heavy_body(causal, x_ref, a_ref, rq, rkv, hq, hk, hv, ho, o_ref,
                oht_s, sc_s, rank_s, h_s):
    _chunked_store(h_s, lambda sl: _ln(x_ref[sl, :]))
    oht_q, hout = _heavy(h_s, rq[:], rkv[:],
                         hq[:], hk[:], hv[:], ho[:], causal, sc_s, rank_s)
    oht_s[:] = oht_q
    _chunked_store(o_ref,
                   lambda sl: a_ref[sl, :] + _dot(oht_s[sl, :], hout, HI))


def _kvprep_body(enc_ref, rkv, o_ref, sc_s, rank_s, e_s):
    _chunked_store(e_s, lambda sl: _ln(enc_ref[sl, :]))
    skv0 = _dot(e_s[:], rkv[:], HI)
    oh_k, _ = _route(skv0, sc_s, rank_s)
    kvv = _dot(oh_k, skv0, HI)
    o_ref[:] = _dot(oh_k, e_s[:], HI) * _sigmoid(kvv)


def _cross_heavy_body(x_ref, a_ref, ks_ref, rq, hq, hk, hv, ho, o_ref,
                      oht_s, sc_s, rank_s, h_s):
    _chunked_store(h_s, lambda sl: _ln(x_ref[sl, :]))
    oht_q, hout = _heavy_cross(h_s, ks_ref[:], rq[:],
                               hq[:], hk[:], hv[:], ho[:], sc_s, rank_s)
    oht_s[:] = oht_q
    _chunked_store(o_ref,
                   lambda sl: a_ref[sl, :] + _dot(oht_s[sl, :], hout, HI))


def _ff_light_body(x_ref, lf1, lf2, o_ref):
    _chunked_store(
        o_ref,
        lambda sl: x_ref[sl, :] + _dot(
            jax.nn.gelu(_dot(_ln(x_ref[sl, :]), lf1[:])), lf2[:]))


def _ff_heavy_body(x_ref, y_ref, rf, hf1, hf2, o_ref, oht_s, sc_s, rank_s,
                   h_s):
    _chunked_store(h_s, lambda sl: _ln(x_ref[sl, :]))
    fs0 = _dot(h_s[:], rf[:], HI)
    oh, oht = _route(fs0, sc_s, rank_s)
    fv = _dot(oh, fs0, HI)
    sel = _dot(oh, h_s[:], HI)
    hff = _dot(jax.nn.gelu(_dot(sel, hf1[:])), hf2[:]) * _sigmoid(fv)
    oht_s[:] = oht
    _chunked_store(o_ref,
                   lambda sl: y_ref[sl, :] + _dot(oht_s[sl, :], hff, HI))


_TC_PARAMS = pltpu.CompilerParams(vmem_limit_bytes=128 * 1024 * 1024)
_OUT_TD = jax.ShapeDtypeStruct((T, D), F32)


_LIGHT_SCRATCH = [pltpu.VMEM((T, LD), F32) for _ in range(4)]
_HEAVY_SCRATCH = [pltpu.VMEM((T, KH), F32),
                  pltpu.VMEM((T, 1), F32), pltpu.VMEM((T, 1), F32),
                  pltpu.VMEM((T, D), F32)]


def _self_block(x, p, causal):
    a = pl.pallas_call(
        functools.partial(_light_body, causal, False),
        out_shape=_OUT_TD, scratch_shapes=list(_LIGHT_SCRATCH),
        compiler_params=_TC_PARAMS)(
        x, p['lq'], p['lk'], p['lv'], p['lo'])
    return pl.pallas_call(
        functools.partial(_heavy_body, causal),
        out_shape=_OUT_TD, scratch_shapes=list(_HEAVY_SCRATCH),
        compiler_params=_TC_PARAMS)(
        x, a, p['rq'].reshape(D, 1), p['rkv'].reshape(D, 1),
        p['hq'], p['hk'], p['hv'], p['ho'])


def _cross_block(x, enc, p):
    a = pl.pallas_call(
        functools.partial(_light_body, False, True),
        out_shape=_OUT_TD, scratch_shapes=list(_LIGHT_SCRATCH),
        compiler_params=_TC_PARAMS)(
        x, enc, p['clq'], p['clk'], p['clv'], p['clo'])
    ks = pl.pallas_call(
        _kvprep_body, out_shape=jax.ShapeDtypeStruct((KH, D), F32),
        scratch_shapes=[pltpu.VMEM((T, 1), F32), pltpu.VMEM((T, 1), F32),
                        pltpu.VMEM((T, D), F32)],
        compiler_params=_TC_PARAMS)(
        enc, p['crkv'].reshape(D, 1))
    return pl.pallas_call(
        _cross_heavy_body,
        out_shape=_OUT_TD, scratch_shapes=list(_HEAVY_SCRATCH),
        compiler_params=_TC_PARAMS)(
        x, a, ks, p['crq'].reshape(D, 1),
        p['cq'], p['ck'], p['cv'], p['co'])


def _cond_ff(x, p):
    y = pl.pallas_call(
        _ff_light_body, out_shape=_OUT_TD,
        compiler_params=_TC_PARAMS)(x, p['lf1'], p['lf2'])
    return pl.pallas_call(
        _ff_heavy_body, out_shape=_OUT_TD,
        scratch_shapes=[pltpu.VMEM((T, KH), F32),
                        pltpu.VMEM((T, 1), F32), pltpu.VMEM((T, 1), F32),
                        pltpu.VMEM((T, D), F32)],
        compiler_params=_TC_PARAMS)(
        x, y, p['rf'].reshape(D, 1), p['hf1'], p['hf2'])


def _lm_head_body(x_ref, w_ref, b_ref, o_ref):
    o_ref[:] = jnp.dot(x_ref[:], w_ref[:],
                       preferred_element_type=F32) + b_ref[:]


def _lm_head(y, w, b):
    return pl.pallas_call(
        _lm_head_body,
        grid=(VOCAB // 128,),
        in_specs=[pl.BlockSpec((T, D), lambda i: (0, 0)),
                  pl.BlockSpec((D, 128), lambda i: (0, i)),
                  pl.BlockSpec((1, 128), lambda i: (0, i))],
        out_specs=pl.BlockSpec((T, 128), lambda i: (0, i)),
        out_shape=jax.ShapeDtypeStruct((T, VOCAB), F32),
        compiler_params=_TC_PARAMS,
    )(y, w, b.reshape(1, VOCAB))


def _embed_gather(table, ids):
    info = plsc.get_sparse_core_info()
    nc, ns = info.num_cores, info.num_subcores
    nw = nc * ns
    bpw = T // nw
    mesh = plsc.VectorSubcoreMesh(core_axis_name="c", subcore_axis_name="s")

    @functools.partial(
        pl.kernel, mesh=mesh,
        out_type=jax.ShapeDtypeStruct((T, D), F32),
        scratch_types=[pltpu.VMEM((bpw,), jnp.int32),
                       pltpu.VMEM((bpw, D), F32),
                       pltpu.SemaphoreType.DMA])
    def k(table_hbm, idx_hbm, out_hbm, idx_v, rows_v, sem):
        wid = lax.axis_index("s") * nc + lax.axis_index("c")
        base = wid * bpw
        pltpu.sync_copy(idx_hbm.at[pl.ds(base, bpw)], idx_v)
        pltpu.async_copy(table_hbm.at[idx_v], rows_v, sem).wait()
        pltpu.sync_copy(rows_v, out_hbm.at[pl.ds(base, bpw)])

    return k(table, ids)


def kernel(params, input_ids, decoder_input_ids):
    p = params
    x = _embed_gather(p['enc_embed'], input_ids.reshape(T))
    for lp in p['enc_layers']:
        x = _self_block(x, lp, causal=False)
        x = _cond_ff(x, lp)
    enc = x
    y = _embed_gather(p['dec_embed'], decoder_input_ids.reshape(T))
    for lp in p['dec_layers']:
        y = _self_block(y, lp, causal=True)
        y = _cross_block(y, enc, lp)
        y = _cond_ff(y, lp)
    logits = _lm_head(y, p['lm_w'], p['lm_b'])
    return logits.reshape(1, T, VOCAB)
